# baseline (speedup 1.0000x reference)
import functools

import jax
import jax.numpy as jnp
from jax import lax
from jax.experimental import pallas as pl
from jax.experimental.pallas import tpu as pltpu
from jax.experimental.pallas import tpu_sc as plsc

_N = 10000
_NH = 5000
_ACC = _NH + 8
_NG = 8
_CE = 1600
_HO = 632


def _edge_body(e_ref, w_ref, m_ref, o_ref):
    res = (
        jnp.dot(e_ref[...], w_ref[...], preferred_element_type=jnp.float32)
        * m_ref[...]
    )
    o_ref[...] = res


def _edge_stage(m_ji, e_rbf, W_edge):
    E, D = m_ji.shape
    R = e_rbf.shape[1]
    BE = 3200
    return pl.pallas_call(
        _edge_body,
        grid=(E // BE,),
        in_specs=[
            pl.BlockSpec((BE, R), lambda i: (i, 0)),
            pl.BlockSpec((R, D), lambda i: (0, 0)),
            pl.BlockSpec((BE, D), lambda i: (i, 0)),
        ],
        out_specs=pl.BlockSpec((BE, D), lambda i: (i, 0)),
        out_shape=jax.ShapeDtypeStruct((E, D), jnp.float32),
    )(e_rbf, W_edge, m_ji)


_BS = 8000


def _seg_body_tc(dst_ref, prod_ref, o_ref):
    @pl.when(pl.program_id(0) == 0)
    def _init():
        o_ref[...] = jnp.zeros_like(o_ref)

    def _edge(e, _):
        d = dst_ref[e // 8, e % 8]
        o_ref[pl.ds(d, 1), :] += prod_ref[pl.ds(e, 1), :]
        return 0
    lax.fori_loop(0, _BS, _edge, 0, unroll=8)


def _scatter_stage(prod, dst):
    E, D = prod.shape
    return pl.pallas_call(
        _seg_body_tc,
        grid=(E // _BS,),
        in_specs=[
            pl.BlockSpec((_BS // 8, 8), lambda i: (i, 0),
                         memory_space=pltpu.SMEM),
            pl.BlockSpec((_BS, D), lambda i: (i, 0)),
        ],
        out_specs=pl.BlockSpec((_N, D), lambda i: (0, 0)),
        out_shape=jax.ShapeDtypeStruct((_N, D), jnp.float32),
    )(dst.reshape(E // 8, 8), prod)


def _swish(x):
    return x * jax.nn.sigmoid(x)


def _mlp_body(p_ref, W1_ref, b1_ref, W2_ref, b2_ref, W3_ref,
              b3_ref, Wf_ref, o_ref):
    x = p_ref[...]
    x = _swish(jnp.dot(x, W1_ref[...], preferred_element_type=jnp.float32)
               + b1_ref[...])
    x = _swish(jnp.dot(x, W2_ref[...], preferred_element_type=jnp.float32)
               + b2_ref[...])
    x = _swish(jnp.dot(x, W3_ref[...], preferred_element_type=jnp.float32)
               + b3_ref[...])
    o_ref[...] = jnp.dot(x, Wf_ref[...], preferred_element_type=jnp.float32)


def _mlp_stage(p, W1, b1, W2, b2, W3, b3, W_final):
    D = W1.shape[0]
    BN = 2000
    full = pl.BlockSpec((D, D), lambda i: (0, 0))
    bias = pl.BlockSpec((1, D), lambda i: (0, 0))
    nblk = pl.BlockSpec((BN, D), lambda i: (i, 0))
    return pl.pallas_call(
        _mlp_body,
        grid=(_N // BN,),
        in_specs=[nblk, full, bias, full, bias, full, bias, full],
        out_specs=nblk,
        out_shape=jax.ShapeDtypeStruct((_N, D), jnp.float32),
    )(p, W1, b1.reshape(1, D), W2, b2.reshape(1, D),
      W3, b3.reshape(1, D), W_final)


def kernel(m_ji, e_rbf, nbr_list, num_atoms, W_edge, W1, b1, W2, b2, W3, b3,
           W_final):
    del num_atoms
    prod = _edge_stage(m_ji, e_rbf, W_edge)
    dst = nbr_list[:, 0]
    node_feats = _scatter_stage(prod, dst)
    return _mlp_stage(node_feats, W1, b1, W2, b2, W3, b3, W_final)
